# Initial kernel scaffold; baseline (speedup 1.0000x reference)
import jax
import jax.numpy as jnp
from jax.experimental import pallas as pl


def kernel(feature, edge_index, W, b):
    raise NotImplementedError("write your pallas kernel here")


# speedup vs baseline: 2.9812x; 2.9812x over previous
import functools

import jax
import jax.numpy as jnp
from jax import lax
from jax.experimental import pallas as pl
from jax.experimental.pallas import tpu as pltpu
from jax.experimental.pallas import tpu_sc as plsc

N = 10000
E = 320000
D = 128
NC, NS = 2, 16
NW = NC * NS
K = 128
CPW = 79
EPW = CPW * K
E_PAD = NW * EPW
NP = 10240
RPT = NP // NS
RB = 64
GB = NP // 128

_MESH = plsc.VectorSubcoreMesh(
    core_axis_name="c", subcore_axis_name="s", num_cores=NC, num_subcores=NS)


def _deg_body(src_hbm, dst_hbm, zer_hbm, out_s, out_d,
              sh_s, sh_d, hs_v, hd_v, sidx, didx, mbuf, res_v):
    c = lax.axis_index("c")
    s = lax.axis_index("s")
    wid = c * NS + s
    pltpu.sync_copy(zer_hbm, hs_v)
    pltpu.sync_copy(zer_hbm, hd_v)
    ones = jnp.ones((16,), jnp.float32)

    def step(i, carry):
        base = wid * EPW + i * K
        pltpu.sync_copy(src_hbm.at[pl.ds(base, K)], sidx)
        pltpu.sync_copy(dst_hbm.at[pl.ds(base, K)], didx)
        for j in range(K // 16):
            plsc.addupdate_scatter(hs_v, [sidx[pl.ds(j * 16, 16)]], ones)
            plsc.addupdate_scatter(hd_v, [didx[pl.ds(j * 16, 16)]], ones)
        return carry

    lax.fori_loop(0, CPW, step, 0)
    pltpu.sync_copy(hs_v, sh_s.at[s])
    pltpu.sync_copy(hd_v, sh_d.at[s])
    plsc.subcore_barrier()
    for sh, out in ((sh_s, out_s), (sh_d, out_d)):
        for t in range(NS):
            pltpu.sync_copy(sh.at[t, pl.ds(s * RPT, RPT)], mbuf.at[t])
        for cj in range(RPT // 16):
            tot = mbuf[0, pl.ds(cj * 16, 16)]
            for t in range(1, NS):
                tot = tot + mbuf[t, pl.ds(cj * 16, 16)]
            res_v[pl.ds(cj * 16, 16)] = tot
        pltpu.sync_copy(res_v, out.at[pl.ds(c * NP + s * RPT, RPT)])


_deg_kernel = pl.kernel(
    _deg_body,
    out_type=[jax.ShapeDtypeStruct((NC * NP,), jnp.float32),
              jax.ShapeDtypeStruct((NC * NP,), jnp.float32)],
    mesh=_MESH,
    scratch_types=[
        pltpu.VMEM_SHARED((NS, NP), jnp.float32),
        pltpu.VMEM_SHARED((NS, NP), jnp.float32),
        pltpu.VMEM((NP,), jnp.float32),
        pltpu.VMEM((NP,), jnp.float32),
        pltpu.VMEM((K,), jnp.int32),
        pltpu.VMEM((K,), jnp.int32),
        pltpu.VMEM((NS, RPT), jnp.float32),
        pltpu.VMEM((RPT,), jnp.float32),
    ],
    compiler_params=pltpu.CompilerParams(needs_layout_passes=False),
)


def _edge_body(src_hbm, dst_hbm, h_hbm, zer_hbm, out_acc,
               acc, sidx, didx, rows, buf, sem):
    c = lax.axis_index("c")
    s = lax.axis_index("s")
    wid = c * NS + s
    pltpu.sync_copy(zer_hbm, buf)

    def zstep(j, carry):
        pltpu.sync_copy(buf, acc.at[pl.ds(s * RPT + j * RB, RB)])
        return carry

    lax.fori_loop(0, RPT // RB, zstep, 0)
    plsc.subcore_barrier()

    def step(i, carry):
        base = wid * EPW + i * K
        pltpu.sync_copy(src_hbm.at[pl.ds(base, K)], sidx)
        pltpu.sync_copy(dst_hbm.at[pl.ds(base, K)], didx)
        pltpu.async_copy(h_hbm.at[sidx], rows, sem).wait()
        pltpu.sync_copy(rows, acc.at[didx], add=True)
        return carry

    lax.fori_loop(0, CPW, step, 0)
    plsc.subcore_barrier()

    def wstep(j, carry):
        pltpu.sync_copy(acc.at[pl.ds(s * RPT + j * RB, RB)], buf)
        pltpu.sync_copy(buf, out_acc.at[pl.ds(c * NP + s * RPT + j * RB, RB)])
        return carry

    lax.fori_loop(0, RPT // RB, wstep, 0)


_edge_kernel = pl.kernel(
    _edge_body,
    out_type=jax.ShapeDtypeStruct((NC * NP, D), jnp.float32),
    mesh=_MESH,
    scratch_types=[
        pltpu.VMEM_SHARED((NP, D), jnp.float32),
        pltpu.VMEM((K,), jnp.int32),
        pltpu.VMEM((K,), jnp.int32),
        pltpu.VMEM((K, D), jnp.float32),
        pltpu.VMEM((RB, D), jnp.float32),
        pltpu.SemaphoreType.DMA,
    ],
)


def _scale_body(f_ref, h0_ref, h1_ref, o_ref):
    cnt = h0_ref[0, 0, :] + h1_ref[0, 0, :]
    scale = lax.rsqrt(jnp.maximum(cnt, 1.0))
    o_ref[...] = f_ref[...] * scale[:, None]


def _out_body(a0_ref, a1_ref, h0_ref, h1_ref, w_ref, b_ref, o_ref):
    cnt = h0_ref[0, 0, :] + h1_ref[0, 0, :]
    inv = lax.rsqrt(jnp.maximum(cnt, 1.0))
    x = (a0_ref[...] + a1_ref[...]) * inv[:, None]
    y = jnp.dot(x, w_ref[...], preferred_element_type=jnp.float32)
    o_ref[...] = jnp.maximum(y + b_ref[0:1, :], 0.0)


def kernel(feature, edge_index, W, b):
    src = edge_index[0]
    dst = edge_index[1]
    pad = jnp.full((E_PAD - E,), NP - 1, dtype=jnp.int32)
    src_p = jnp.concatenate([src, pad])
    dst_p = jnp.concatenate([dst, pad])
    feature_p = jnp.pad(feature, ((0, NP - N), (0, 0)))
    zer_deg = jnp.zeros((NP,), dtype=jnp.float32)
    zer_acc = jnp.zeros((RB, D), dtype=jnp.float32)
    b2 = jnp.broadcast_to(b, (8, D))

    hist_s, hist_d = _deg_kernel(src_p, dst_p, zer_deg)
    hist_s3 = hist_s.reshape(NC * GB, 1, 128)
    hist_d3 = hist_d.reshape(NC * GB, 1, 128)

    h = pl.pallas_call(
        _scale_body,
        grid=(GB,),
        in_specs=[pl.BlockSpec((128, D), lambda i: (i, 0)),
                  pl.BlockSpec((1, 1, 128), lambda i: (i, 0, 0)),
                  pl.BlockSpec((1, 1, 128), lambda i: (i + GB, 0, 0))],
        out_specs=pl.BlockSpec((128, D), lambda i: (i, 0)),
        out_shape=jax.ShapeDtypeStruct((NP, D), jnp.float32),
    )(feature_p, hist_s3, hist_s3)

    acc = _edge_kernel(src_p, dst_p, h, zer_acc)

    out = pl.pallas_call(
        _out_body,
        grid=(GB,),
        in_specs=[pl.BlockSpec((128, D), lambda i: (i, 0)),
                  pl.BlockSpec((128, D), lambda i: (i + GB, 0)),
                  pl.BlockSpec((1, 1, 128), lambda i: (i, 0, 0)),
                  pl.BlockSpec((1, 1, 128), lambda i: (i + GB, 0, 0)),
                  pl.BlockSpec((128, D), lambda i: (0, 0)),
                  pl.BlockSpec((8, D), lambda i: (0, 0))],
        out_specs=pl.BlockSpec((128, D), lambda i: (i, 0)),
        out_shape=jax.ShapeDtypeStruct((NP, D), jnp.float32),
    )(acc, acc, hist_d3, hist_d3, W, b2)

    return out[:N]


# speedup vs baseline: 3.2047x; 1.0750x over previous
import jax
import jax.numpy as jnp
from jax import lax
from jax.experimental import pallas as pl
from jax.experimental.pallas import tpu as pltpu
from jax.experimental.pallas import tpu_sc as plsc

N = 10000
E = 320000
D = 128
NC, NS = 2, 16
NW = NC * NS
K = 128
CPW = 80
HC = CPW // 2
EPW = CPW * K
E_PAD = NW * EPW
NP = 10240
RPT = NP // NS
HB = NP // K
HR = HB // NS
GB = NP // 128

_MESH = plsc.VectorSubcoreMesh(
    core_axis_name="c", subcore_axis_name="s", num_cores=NC, num_subcores=NS)


def _deg_body(src2_hbm, dst2_hbm, zer2_hbm, iden_hbm, out_s, out_d,
              sh_s, sh_d, hs_v, hd_v, sidx, didx, iden_v, tbuf):
    c = lax.axis_index("c")
    s = lax.axis_index("s")
    wid = c * NS + s
    pltpu.sync_copy(src2_hbm.at[pl.ds(wid * CPW, CPW)], sidx)
    pltpu.sync_copy(dst2_hbm.at[pl.ds(wid * CPW, CPW)], didx)
    pltpu.sync_copy(zer2_hbm, hs_v)
    pltpu.sync_copy(zer2_hbm, hd_v)
    pltpu.sync_copy(iden_hbm, iden_v)

    @pl.when(s < HB // 8)
    def _():
        pltpu.sync_copy(hs_v.at[pl.ds(s * 8, 8)], sh_s.at[pl.ds(s * 8, 8)])
        pltpu.sync_copy(hd_v.at[pl.ds(s * 8, 8)], sh_d.at[pl.ds(s * 8, 8)])

    ones = jnp.ones((16,), jnp.float32)

    def step(i, carry):
        for j in range(K // 16):
            si = sidx[i, pl.ds(j * 16, 16)]
            plsc.addupdate_scatter(
                hs_v,
                [lax.shift_right_logical(si, 7), lax.bitwise_and(si, 127)],
                ones)
            di = didx[i, pl.ds(j * 16, 16)]
            plsc.addupdate_scatter(
                hd_v,
                [lax.shift_right_logical(di, 7), lax.bitwise_and(di, 127)],
                ones)
        return carry

    lax.fori_loop(0, CPW, step, 0)
    plsc.subcore_barrier()
    pltpu.sync_copy(hs_v, sh_s.at[iden_v], add=True)
    pltpu.sync_copy(hd_v, sh_d.at[iden_v], add=True)
    plsc.subcore_barrier()

    @pl.when(s < HB // 8)
    def _():
        pltpu.sync_copy(sh_s.at[pl.ds(s * 8, 8)], tbuf)
        pltpu.sync_copy(tbuf, out_s.at[pl.ds(c * HB + s * 8, 8)])
        pltpu.sync_copy(sh_d.at[pl.ds(s * 8, 8)], tbuf)
        pltpu.sync_copy(tbuf, out_d.at[pl.ds(c * HB + s * 8, 8)])


_deg_kernel = pl.kernel(
    _deg_body,
    out_type=[jax.ShapeDtypeStruct((NC * HB, 128), jnp.float32),
              jax.ShapeDtypeStruct((NC * HB, 128), jnp.float32)],
    mesh=_MESH,
    scratch_types=[
        pltpu.VMEM_SHARED((HB, 128), jnp.float32),
        pltpu.VMEM_SHARED((HB, 128), jnp.float32),
        pltpu.VMEM((HB, 128), jnp.float32),
        pltpu.VMEM((HB, 128), jnp.float32),
        pltpu.VMEM((CPW, K), jnp.int32),
        pltpu.VMEM((CPW, K), jnp.int32),
        pltpu.VMEM((HB,), jnp.int32),
        pltpu.VMEM((8, 128), jnp.float32),
    ],
    compiler_params=pltpu.CompilerParams(needs_layout_passes=False),
)


def _edge_body(src2_hbm, dst2_hbm, h_hbm, zer2_hbm, out_acc,
               acc, sidx, didx, rows0, rows1, g0, g1):
    c = lax.axis_index("c")
    s = lax.axis_index("s")
    wid = c * NS + s
    pltpu.sync_copy(zer2_hbm, rows0)
    for j in range(RPT // K):
        pltpu.sync_copy(rows0, acc.at[pl.ds(s * RPT + j * K, K)])
    plsc.subcore_barrier()

    for half in range(2):
        rbase = wid * CPW + half * HC
        pltpu.sync_copy(src2_hbm.at[pl.ds(rbase, HC)], sidx)
        pltpu.sync_copy(dst2_hbm.at[pl.ds(rbase, HC)], didx)
        pltpu.async_copy(h_hbm.at[sidx.at[0]], rows0, g0)

        def step2(i2, carry):
            i0 = 2 * i2
            pltpu.async_copy(h_hbm.at[sidx.at[i0 + 1]], rows1, g1)
            pltpu.make_async_copy(h_hbm.at[sidx.at[i0]], rows0, g0).wait()
            pltpu.sync_copy(rows0, acc.at[didx.at[i0]], add=True)

            @pl.when(i0 + 2 < HC)
            def _():
                pltpu.async_copy(h_hbm.at[sidx.at[i0 + 2]], rows0, g0)

            pltpu.make_async_copy(h_hbm.at[sidx.at[i0 + 1]], rows1, g1).wait()
            pltpu.sync_copy(rows1, acc.at[didx.at[i0 + 1]], add=True)
            return carry

        lax.fori_loop(0, HC // 2, step2, 0)

    plsc.subcore_barrier()
    for j in range(RPT // K):
        pltpu.sync_copy(acc.at[pl.ds(s * RPT + j * K, K)], rows0)
        pltpu.sync_copy(rows0, out_acc.at[pl.ds(c * NP + s * RPT + j * K, K)])


_edge_kernel = pl.kernel(
    _edge_body,
    out_type=jax.ShapeDtypeStruct((NC * NP, D), jnp.float32),
    mesh=_MESH,
    scratch_types=[
        pltpu.VMEM_SHARED((NP, D), jnp.float32),
        pltpu.VMEM((HC, K), jnp.int32),
        pltpu.VMEM((HC, K), jnp.int32),
        pltpu.VMEM((K, D), jnp.float32),
        pltpu.VMEM((K, D), jnp.float32),
        pltpu.SemaphoreType.DMA,
        pltpu.SemaphoreType.DMA,
    ],
)


def _scale_body(f_ref, h0_ref, h1_ref, o_ref):
    cnt = h0_ref[0, 0, :] + h1_ref[0, 0, :]
    scale = lax.rsqrt(jnp.maximum(cnt, 1.0))
    o_ref[...] = f_ref[...] * scale[:, None]


def _out_body(a0_ref, a1_ref, h0_ref, h1_ref, w_ref, b_ref, o_ref):
    cnt = h0_ref[0, 0, :] + h1_ref[0, 0, :]
    inv = lax.rsqrt(jnp.maximum(cnt, 1.0))
    x = (a0_ref[...] + a1_ref[...]) * inv[:, None]
    y = jnp.dot(x, w_ref[...], preferred_element_type=jnp.float32)
    o_ref[...] = jnp.maximum(y + b_ref[0:1, :], 0.0)


def kernel(feature, edge_index, W, b):
    src = edge_index[0]
    dst = edge_index[1]
    pad = jnp.full((E_PAD - E,), NP - 1, dtype=jnp.int32)
    src2 = jnp.concatenate([src, pad]).reshape(E_PAD // K, K)
    dst2 = jnp.concatenate([dst, pad]).reshape(E_PAD // K, K)
    feature_p = jnp.pad(feature, ((0, NP - N), (0, 0)))
    zer_h = jnp.zeros((HB, 128), dtype=jnp.float32)
    zer_r = jnp.zeros((K, D), dtype=jnp.float32)
    iden = jnp.arange(HB, dtype=jnp.int32)
    b2 = jnp.broadcast_to(b, (8, D))

    hist_s, hist_d = _deg_kernel(src2, dst2, zer_h, iden)
    hist_s3 = hist_s.reshape(NC * GB, 1, 128)
    hist_d3 = hist_d.reshape(NC * GB, 1, 128)

    h = pl.pallas_call(
        _scale_body,
        grid=(GB,),
        in_specs=[pl.BlockSpec((128, D), lambda i: (i, 0)),
                  pl.BlockSpec((1, 1, 128), lambda i: (i, 0, 0)),
                  pl.BlockSpec((1, 1, 128), lambda i: (i + GB, 0, 0))],
        out_specs=pl.BlockSpec((128, D), lambda i: (i, 0)),
        out_shape=jax.ShapeDtypeStruct((NP, D), jnp.float32),
    )(feature_p, hist_s3, hist_s3)

    acc = _edge_kernel(src2, dst2, h, zer_r)

    out = pl.pallas_call(
        _out_body,
        grid=(GB,),
        in_specs=[pl.BlockSpec((128, D), lambda i: (i, 0)),
                  pl.BlockSpec((128, D), lambda i: (i + GB, 0)),
                  pl.BlockSpec((1, 1, 128), lambda i: (i, 0, 0)),
                  pl.BlockSpec((1, 1, 128), lambda i: (i + GB, 0, 0)),
                  pl.BlockSpec((128, D), lambda i: (0, 0)),
                  pl.BlockSpec((8, D), lambda i: (0, 0))],
        out_specs=pl.BlockSpec((128, D), lambda i: (i, 0)),
        out_shape=jax.ShapeDtypeStruct((NP, D), jnp.float32),
    )(acc, acc, hist_d3, hist_d3, W, b2)

    return out[:N]


# speedup vs baseline: 3.8894x; 1.2137x over previous
import jax
import jax.numpy as jnp
from jax import lax
from jax.experimental import pallas as pl
from jax.experimental.pallas import tpu as pltpu
from jax.experimental.pallas import tpu_sc as plsc

N = 10000
E = 320000
D = 128
NC, NS = 2, 16
K = 128
CPW0 = 128
CPW1 = 32
IB = 32
E_PAD = NS * (CPW0 + CPW1) * K
NP = 10240
RPT = NP // NS
HB = NP // K
GB = NP // 1024

_MESH = plsc.VectorSubcoreMesh(
    core_axis_name="c", subcore_axis_name="s", num_cores=NC, num_subcores=NS)


def _worker_layout(c, s):
    rbase = jnp.where(c == 0, s * CPW0, NS * CPW0 + s * CPW1)
    nblk = jnp.where(c == 0, CPW0 // IB, CPW1 // IB)
    return rbase, nblk


def _deg_body(src2_hbm, dst2_hbm, zer2_hbm, iden_hbm, out_s, out_d,
              sh_s, sh_d, hs_v, hd_v, sidx, didx, iden_v, tbuf):
    c = lax.axis_index("c")
    s = lax.axis_index("s")
    rbase, nblk = _worker_layout(c, s)
    pltpu.sync_copy(zer2_hbm, hs_v)
    pltpu.sync_copy(zer2_hbm, hd_v)
    pltpu.sync_copy(iden_hbm, iden_v)

    @pl.when(s < HB // 8)
    def _():
        pltpu.sync_copy(hs_v.at[pl.ds(s * 8, 8)], sh_s.at[pl.ds(s * 8, 8)])
        pltpu.sync_copy(hd_v.at[pl.ds(s * 8, 8)], sh_d.at[pl.ds(s * 8, 8)])

    ones = jnp.ones((16,), jnp.float32)

    def blk(bi, carry):
        pltpu.sync_copy(src2_hbm.at[pl.ds(rbase + bi * IB, IB)], sidx)
        pltpu.sync_copy(dst2_hbm.at[pl.ds(rbase + bi * IB, IB)], didx)

        def step(i, carry2):
            for j in range(K // 16):
                si = sidx[i, pl.ds(j * 16, 16)]
                plsc.addupdate_scatter(
                    hs_v,
                    [lax.shift_right_logical(si, 7), lax.bitwise_and(si, 127)],
                    ones)
                di = didx[i, pl.ds(j * 16, 16)]
                plsc.addupdate_scatter(
                    hd_v,
                    [lax.shift_right_logical(di, 7), lax.bitwise_and(di, 127)],
                    ones)
            return carry2

        lax.fori_loop(0, IB, step, 0)
        return carry

    lax.fori_loop(0, nblk, blk, 0)
    plsc.subcore_barrier()
    pltpu.sync_copy(hs_v, sh_s.at[iden_v], add=True)
    pltpu.sync_copy(hd_v, sh_d.at[iden_v], add=True)
    plsc.subcore_barrier()

    @pl.when(s < HB // 8)
    def _():
        pltpu.sync_copy(sh_s.at[pl.ds(s * 8, 8)], tbuf)
        pltpu.sync_copy(tbuf, out_s.at[pl.ds(c * HB + s * 8, 8)])
        pltpu.sync_copy(sh_d.at[pl.ds(s * 8, 8)], tbuf)
        pltpu.sync_copy(tbuf, out_d.at[pl.ds(c * HB + s * 8, 8)])


_deg_kernel = pl.kernel(
    _deg_body,
    out_type=[jax.ShapeDtypeStruct((NC * HB, 128), jnp.float32),
              jax.ShapeDtypeStruct((NC * HB, 128), jnp.float32)],
    mesh=_MESH,
    scratch_types=[
        pltpu.VMEM_SHARED((HB, 128), jnp.float32),
        pltpu.VMEM_SHARED((HB, 128), jnp.float32),
        pltpu.VMEM((HB, 128), jnp.float32),
        pltpu.VMEM((HB, 128), jnp.float32),
        pltpu.VMEM((IB, K), jnp.int32),
        pltpu.VMEM((IB, K), jnp.int32),
        pltpu.VMEM((HB,), jnp.int32),
        pltpu.VMEM((8, 128), jnp.float32),
    ],
    compiler_params=pltpu.CompilerParams(needs_layout_passes=False),
)


def _edge_body(src2_hbm, dst2_hbm, h_hbm, zer2_hbm, out_acc,
               acc, sidx, didx, rows0, rows1, g0, g1):
    c = lax.axis_index("c")
    s = lax.axis_index("s")
    rbase, nblk = _worker_layout(c, s)
    pltpu.sync_copy(zer2_hbm, rows0)
    for j in range(RPT // K):
        pltpu.sync_copy(rows0, acc.at[pl.ds(s * RPT + j * K, K)])
    plsc.subcore_barrier()

    def blk(bi, carry):
        pltpu.sync_copy(src2_hbm.at[pl.ds(rbase + bi * IB, IB)], sidx)
        pltpu.sync_copy(dst2_hbm.at[pl.ds(rbase + bi * IB, IB)], didx)
        pltpu.async_copy(h_hbm.at[sidx.at[0]], rows0, g0)

        def step2(i2, carry2):
            i0 = 2 * i2
            pltpu.async_copy(h_hbm.at[sidx.at[i0 + 1]], rows1, g1)
            pltpu.make_async_copy(h_hbm.at[sidx.at[i0]], rows0, g0).wait()
            pltpu.sync_copy(rows0, acc.at[didx.at[i0]], add=True)

            @pl.when(i0 + 2 < IB)
            def _():
                pltpu.async_copy(h_hbm.at[sidx.at[i0 + 2]], rows0, g0)

            pltpu.make_async_copy(h_hbm.at[sidx.at[i0 + 1]], rows1, g1).wait()
            pltpu.sync_copy(rows1, acc.at[didx.at[i0 + 1]], add=True)
            return carry2

        lax.fori_loop(0, IB // 2, step2, 0)
        return carry

    lax.fori_loop(0, nblk, blk, 0)
    plsc.subcore_barrier()
    for j in range(RPT // K):
        pltpu.sync_copy(acc.at[pl.ds(s * RPT + j * K, K)], rows0)
        pltpu.sync_copy(rows0, out_acc.at[pl.ds(c * NP + s * RPT + j * K, K)])


_edge_kernel = pl.kernel(
    _edge_body,
    out_type=jax.ShapeDtypeStruct((NC * NP, D), jnp.float32),
    mesh=_MESH,
    scratch_types=[
        pltpu.VMEM_SHARED((NP, D), jnp.float32),
        pltpu.VMEM((IB, K), jnp.int32),
        pltpu.VMEM((IB, K), jnp.int32),
        pltpu.VMEM((K, D), jnp.float32),
        pltpu.VMEM((K, D), jnp.float32),
        pltpu.SemaphoreType.DMA,
        pltpu.SemaphoreType.DMA,
    ],
)


def _scale_body(f_ref, h0_ref, h1_ref, o_ref):
    cnt = h0_ref[0, 0, :] + h1_ref[0, 0, :]
    scale = lax.rsqrt(jnp.maximum(cnt, 1.0))
    o_ref[...] = f_ref[...] * scale[:, None]


def _out_body(a0_ref, a1_ref, h0_ref, h1_ref, w_ref, b_ref, o_ref):
    cnt = h0_ref[0, 0, :] + h1_ref[0, 0, :]
    inv = lax.rsqrt(jnp.maximum(cnt, 1.0))
    x = (a0_ref[...] + a1_ref[...]) * inv[:, None]
    y = jnp.dot(x, w_ref[...], preferred_element_type=jnp.float32)
    o_ref[...] = jnp.maximum(y + b_ref[0:1, :], 0.0)


def kernel(feature, edge_index, W, b):
    src = edge_index[0]
    dst = edge_index[1]
    pad = jnp.full((E_PAD - E,), NP - 1, dtype=jnp.int32)
    src2 = jnp.concatenate([src, pad]).reshape(E_PAD // K, K)
    dst2 = jnp.concatenate([dst, pad]).reshape(E_PAD // K, K)
    feature_p = jnp.pad(feature, ((0, NP - N), (0, 0)))
    zer_h = jnp.zeros((HB, 128), dtype=jnp.float32)
    zer_r = jnp.zeros((K, D), dtype=jnp.float32)
    iden = jnp.arange(HB, dtype=jnp.int32)
    b2 = jnp.broadcast_to(b, (8, D))

    hist_s, hist_d = _deg_kernel(src2, dst2, zer_h, iden)
    hist_s3 = hist_s.reshape(NC * GB, 1, 1024)
    hist_d3 = hist_d.reshape(NC * GB, 1, 1024)

    h = pl.pallas_call(
        _scale_body,
        grid=(GB,),
        in_specs=[pl.BlockSpec((1024, D), lambda i: (i, 0)),
                  pl.BlockSpec((1, 1, 1024), lambda i: (i, 0, 0)),
                  pl.BlockSpec((1, 1, 1024), lambda i: (i + GB, 0, 0))],
        out_specs=pl.BlockSpec((1024, D), lambda i: (i, 0)),
        out_shape=jax.ShapeDtypeStruct((NP, D), jnp.float32),
    )(feature_p, hist_s3, hist_s3)

    acc = _edge_kernel(src2, dst2, h, zer_r)

    out = pl.pallas_call(
        _out_body,
        grid=(GB,),
        in_specs=[pl.BlockSpec((1024, D), lambda i: (i, 0)),
                  pl.BlockSpec((1024, D), lambda i: (i + GB, 0)),
                  pl.BlockSpec((1, 1, 1024), lambda i: (i, 0, 0)),
                  pl.BlockSpec((1, 1, 1024), lambda i: (i + GB, 0, 0)),
                  pl.BlockSpec((128, D), lambda i: (0, 0)),
                  pl.BlockSpec((8, D), lambda i: (0, 0))],
        out_specs=pl.BlockSpec((1024, D), lambda i: (i, 0)),
        out_shape=jax.ShapeDtypeStruct((NP, D), jnp.float32),
    )(acc, acc, hist_d3, hist_d3, W, b2)

    return out[:N]


# speedup vs baseline: 3.8916x; 1.0006x over previous
import jax
import jax.numpy as jnp
from jax import lax
from jax.experimental import pallas as pl
from jax.experimental.pallas import tpu as pltpu
from jax.experimental.pallas import tpu_sc as plsc

N = 10000
E = 320000
D = 128
NC, NS = 2, 16
K = 128
CPW0 = 128
CPW1 = 32
IB = 32
E_PAD = NS * (CPW0 + CPW1) * K
NP = 10240
RPT = NP // NS
HB = NP // K
GB = NP // 1024

_MESH = plsc.VectorSubcoreMesh(
    core_axis_name="c", subcore_axis_name="s", num_cores=NC, num_subcores=NS)


def _worker_layout(c, s):
    rbase = jnp.where(c == 0, s * CPW0, NS * CPW0 + s * CPW1)
    nblk = jnp.where(c == 0, CPW0 // IB, CPW1 // IB)
    return rbase, nblk


def _deg_body(src2_hbm, dst2_hbm, zer2_hbm, iden_hbm, out_s, out_d,
              sh_s, sh_d, hs_v, hd_v, sidx, didx, iden_v, tbuf):
    c = lax.axis_index("c")
    s = lax.axis_index("s")
    rbase, nblk = _worker_layout(c, s)
    pltpu.sync_copy(zer2_hbm, hs_v)
    pltpu.sync_copy(zer2_hbm, hd_v)
    pltpu.sync_copy(iden_hbm, iden_v)

    @pl.when(s < HB // 8)
    def _():
        pltpu.sync_copy(hs_v.at[pl.ds(s * 8, 8)], sh_s.at[pl.ds(s * 8, 8)])
        pltpu.sync_copy(hd_v.at[pl.ds(s * 8, 8)], sh_d.at[pl.ds(s * 8, 8)])

    ones = jnp.ones((16,), jnp.float32)

    def blk(bi, carry):
        pltpu.sync_copy(src2_hbm.at[pl.ds(rbase + bi * IB, IB)], sidx)
        pltpu.sync_copy(dst2_hbm.at[pl.ds(rbase + bi * IB, IB)], didx)

        def step(i, carry2):
            for j in range(K // 16):
                si = sidx[i, pl.ds(j * 16, 16)]
                plsc.addupdate_scatter(
                    hs_v,
                    [lax.shift_right_logical(si, 7), lax.bitwise_and(si, 127)],
                    ones)
                di = didx[i, pl.ds(j * 16, 16)]
                plsc.addupdate_scatter(
                    hd_v,
                    [lax.shift_right_logical(di, 7), lax.bitwise_and(di, 127)],
                    ones)
            return carry2

        lax.fori_loop(0, IB, step, 0)
        return carry

    lax.fori_loop(0, nblk, blk, 0)
    plsc.subcore_barrier()
    pltpu.sync_copy(hs_v, sh_s.at[iden_v], add=True)
    pltpu.sync_copy(hd_v, sh_d.at[iden_v], add=True)
    plsc.subcore_barrier()

    @pl.when(s < HB // 8)
    def _():
        pltpu.sync_copy(sh_s.at[pl.ds(s * 8, 8)], tbuf)
        pltpu.sync_copy(tbuf, out_s.at[pl.ds(c * HB + s * 8, 8)])
        pltpu.sync_copy(sh_d.at[pl.ds(s * 8, 8)], tbuf)
        pltpu.sync_copy(tbuf, out_d.at[pl.ds(c * HB + s * 8, 8)])


_deg_kernel = pl.kernel(
    _deg_body,
    out_type=[jax.ShapeDtypeStruct((NC * HB, 128), jnp.float32),
              jax.ShapeDtypeStruct((NC * HB, 128), jnp.float32)],
    mesh=_MESH,
    scratch_types=[
        pltpu.VMEM_SHARED((HB, 128), jnp.float32),
        pltpu.VMEM_SHARED((HB, 128), jnp.float32),
        pltpu.VMEM((HB, 128), jnp.float32),
        pltpu.VMEM((HB, 128), jnp.float32),
        pltpu.VMEM((IB, K), jnp.int32),
        pltpu.VMEM((IB, K), jnp.int32),
        pltpu.VMEM((HB,), jnp.int32),
        pltpu.VMEM((8, 128), jnp.float32),
    ],
    compiler_params=pltpu.CompilerParams(needs_layout_passes=False),
)


def _edge_body(src2_hbm, dst2_hbm, h_hbm, zer2_hbm, out_acc,
               acc, sidx, didx, rows0, rows1, g0, g1):
    c = lax.axis_index("c")
    s = lax.axis_index("s")
    rbase, nblk = _worker_layout(c, s)
    with jax.named_scope("zero_acc"):
        pltpu.sync_copy(zer2_hbm, rows0)
        for j in range(RPT // K):
            pltpu.sync_copy(rows0, acc.at[pl.ds(s * RPT + j * K, K)])
        plsc.subcore_barrier()

    def blk(bi, carry):
        pltpu.sync_copy(src2_hbm.at[pl.ds(rbase + bi * IB, IB)], sidx)
        pltpu.sync_copy(dst2_hbm.at[pl.ds(rbase + bi * IB, IB)], didx)
        pltpu.async_copy(h_hbm.at[sidx.at[0]], rows0, g0)

        def step2(i2, carry2):
            i0 = 2 * i2
            pltpu.async_copy(h_hbm.at[sidx.at[i0 + 1]], rows1, g1)
            pltpu.make_async_copy(h_hbm.at[sidx.at[i0]], rows0, g0).wait()
            pltpu.sync_copy(rows0, acc.at[didx.at[i0]], add=True)

            @pl.when(i0 + 2 < IB)
            def _():
                pltpu.async_copy(h_hbm.at[sidx.at[i0 + 2]], rows0, g0)

            pltpu.make_async_copy(h_hbm.at[sidx.at[i0 + 1]], rows1, g1).wait()
            pltpu.sync_copy(rows1, acc.at[didx.at[i0 + 1]], add=True)
            return carry2

        lax.fori_loop(0, IB // 2, step2, 0)
        return carry

    with jax.named_scope("chunks"):
        lax.fori_loop(0, nblk, blk, 0)
        plsc.subcore_barrier()
    with jax.named_scope("writeback"):
        for j in range(RPT // K):
            pltpu.sync_copy(acc.at[pl.ds(s * RPT + j * K, K)], rows0)
            pltpu.sync_copy(
                rows0, out_acc.at[pl.ds(c * NP + s * RPT + j * K, K)])


_edge_kernel = pl.kernel(
    _edge_body,
    out_type=jax.ShapeDtypeStruct((NC * NP, D), jnp.float32),
    mesh=_MESH,
    scratch_types=[
        pltpu.VMEM_SHARED((NP, D), jnp.float32),
        pltpu.VMEM((IB, K), jnp.int32),
        pltpu.VMEM((IB, K), jnp.int32),
        pltpu.VMEM((K, D), jnp.float32),
        pltpu.VMEM((K, D), jnp.float32),
        pltpu.SemaphoreType.DMA,
        pltpu.SemaphoreType.DMA,
    ],
)


def _scale_body(f_ref, h0_ref, h1_ref, o_ref):
    cnt = h0_ref[0, 0, :] + h1_ref[0, 0, :]
    scale = lax.rsqrt(jnp.maximum(cnt, 1.0))
    o_ref[...] = f_ref[...] * scale[:, None]


def _out_body(a0_ref, a1_ref, h0_ref, h1_ref, w_ref, b_ref, o_ref):
    cnt = h0_ref[0, 0, :] + h1_ref[0, 0, :]
    inv = lax.rsqrt(jnp.maximum(cnt, 1.0))
    x = (a0_ref[...] + a1_ref[...]) * inv[:, None]
    y = jnp.dot(x, w_ref[...], preferred_element_type=jnp.float32)
    o_ref[...] = jnp.maximum(y + b_ref[0:1, :], 0.0)


def kernel(feature, edge_index, W, b):
    src = edge_index[0]
    dst = edge_index[1]
    pad = jnp.full((E_PAD - E,), NP - 1, dtype=jnp.int32)
    src2 = jnp.concatenate([src, pad]).reshape(E_PAD // K, K)
    dst2 = jnp.concatenate([dst, pad]).reshape(E_PAD // K, K)
    feature_p = jnp.pad(feature, ((0, NP - N), (0, 0)))
    zer_h = jnp.zeros((HB, 128), dtype=jnp.float32)
    zer_r = jnp.zeros((K, D), dtype=jnp.float32)
    iden = jnp.arange(HB, dtype=jnp.int32)
    b2 = jnp.broadcast_to(b, (8, D))

    hist_s, hist_d = _deg_kernel(src2, dst2, zer_h, iden)
    hist_s3 = hist_s.reshape(NC * GB, 1, 1024)
    hist_d3 = hist_d.reshape(NC * GB, 1, 1024)

    h = pl.pallas_call(
        _scale_body,
        grid=(GB,),
        in_specs=[pl.BlockSpec((1024, D), lambda i: (i, 0)),
                  pl.BlockSpec((1, 1, 1024), lambda i: (i, 0, 0)),
                  pl.BlockSpec((1, 1, 1024), lambda i: (i + GB, 0, 0))],
        out_specs=pl.BlockSpec((1024, D), lambda i: (i, 0)),
        out_shape=jax.ShapeDtypeStruct((NP, D), jnp.float32),
    )(feature_p, hist_s3, hist_s3)

    acc = _edge_kernel(src2, dst2, h, zer_r)

    out = pl.pallas_call(
        _out_body,
        grid=(GB,),
        in_specs=[pl.BlockSpec((1024, D), lambda i: (i, 0)),
                  pl.BlockSpec((1024, D), lambda i: (i + GB, 0)),
                  pl.BlockSpec((1, 1, 1024), lambda i: (i, 0, 0)),
                  pl.BlockSpec((1, 1, 1024), lambda i: (i + GB, 0, 0)),
                  pl.BlockSpec((128, D), lambda i: (0, 0)),
                  pl.BlockSpec((8, D), lambda i: (0, 0))],
        out_specs=pl.BlockSpec((1024, D), lambda i: (i, 0)),
        out_shape=jax.ShapeDtypeStruct((NP, D), jnp.float32),
    )(acc, acc, hist_d3, hist_d3, W, b2)

    return out[:N]


# speedup vs baseline: 4.1373x; 1.0631x over previous
import jax
import jax.numpy as jnp
from jax import lax
from jax.experimental import pallas as pl
from jax.experimental.pallas import tpu as pltpu
from jax.experimental.pallas import tpu_sc as plsc

N = 10000
E = 320000
D = 128
NC, NS = 2, 16
K = 128
CPW0 = 144
CPW1 = 16
IB = 16
DCW = 80
E_PAD = NS * (CPW0 + CPW1) * K
NP = 10240
RPT = NP // NS
HB = NP // K
GB = NP // 1024

_MESH = plsc.VectorSubcoreMesh(
    core_axis_name="c", subcore_axis_name="s", num_cores=NC, num_subcores=NS)


def _worker_layout(c, s):
    rbase = jnp.where(c == 0, s * CPW0, NS * CPW0 + s * CPW1)
    nblk = jnp.where(c == 0, CPW0 // IB, CPW1 // IB)
    return rbase, nblk


def _deg_body(src2_hbm, dst2_hbm, zer2_hbm, iden_hbm, out_s, out_d,
              sh_s, sh_d, hs_v, hd_v, sidx, didx, iden_v, tbuf):
    c = lax.axis_index("c")
    s = lax.axis_index("s")
    rbase = (c * NS + s) * DCW
    nblk = DCW // IB
    pltpu.sync_copy(zer2_hbm, hs_v)
    pltpu.sync_copy(zer2_hbm, hd_v)
    pltpu.sync_copy(iden_hbm, iden_v)

    @pl.when(s < HB // 8)
    def _():
        pltpu.sync_copy(hs_v.at[pl.ds(s * 8, 8)], sh_s.at[pl.ds(s * 8, 8)])
        pltpu.sync_copy(hd_v.at[pl.ds(s * 8, 8)], sh_d.at[pl.ds(s * 8, 8)])

    ones = jnp.ones((16,), jnp.float32)

    def blk(bi, carry):
        pltpu.sync_copy(src2_hbm.at[pl.ds(rbase + bi * IB, IB)], sidx)
        pltpu.sync_copy(dst2_hbm.at[pl.ds(rbase + bi * IB, IB)], didx)

        def step(i, carry2):
            for j in range(K // 16):
                si = sidx[i, pl.ds(j * 16, 16)]
                plsc.addupdate_scatter(
                    hs_v,
                    [lax.shift_right_logical(si, 7), lax.bitwise_and(si, 127)],
                    ones)
                di = didx[i, pl.ds(j * 16, 16)]
                plsc.addupdate_scatter(
                    hd_v,
                    [lax.shift_right_logical(di, 7), lax.bitwise_and(di, 127)],
                    ones)
            return carry2

        lax.fori_loop(0, IB, step, 0)
        return carry

    lax.fori_loop(0, nblk, blk, 0)
    plsc.subcore_barrier()
    pltpu.sync_copy(hs_v, sh_s.at[iden_v], add=True)
    pltpu.sync_copy(hd_v, sh_d.at[iden_v], add=True)
    plsc.subcore_barrier()

    @pl.when(s < HB // 8)
    def _():
        pltpu.sync_copy(sh_s.at[pl.ds(s * 8, 8)], tbuf)
        pltpu.sync_copy(tbuf, out_s.at[pl.ds(c * HB + s * 8, 8)])
        pltpu.sync_copy(sh_d.at[pl.ds(s * 8, 8)], tbuf)
        pltpu.sync_copy(tbuf, out_d.at[pl.ds(c * HB + s * 8, 8)])


_deg_kernel = pl.kernel(
    _deg_body,
    out_type=[jax.ShapeDtypeStruct((NC * HB, 128), jnp.float32),
              jax.ShapeDtypeStruct((NC * HB, 128), jnp.float32)],
    mesh=_MESH,
    scratch_types=[
        pltpu.VMEM_SHARED((HB, 128), jnp.float32),
        pltpu.VMEM_SHARED((HB, 128), jnp.float32),
        pltpu.VMEM((HB, 128), jnp.float32),
        pltpu.VMEM((HB, 128), jnp.float32),
        pltpu.VMEM((IB, K), jnp.int32),
        pltpu.VMEM((IB, K), jnp.int32),
        pltpu.VMEM((HB,), jnp.int32),
        pltpu.VMEM((8, 128), jnp.float32),
    ],
    compiler_params=pltpu.CompilerParams(needs_layout_passes=False),
)


def _edge_body(src2_hbm, dst2_hbm, h_hbm, zer2_hbm, out_acc,
               acc, sidx, didx, rows0, rows1, g0, g1, s0, s1):
    c = lax.axis_index("c")
    s = lax.axis_index("s")
    rbase, nblk = _worker_layout(c, s)
    with jax.named_scope("zero_acc"):
        pltpu.sync_copy(zer2_hbm, rows0)
        for j in range(RPT // K):
            pltpu.sync_copy(rows0, acc.at[pl.ds(s * RPT + j * K, K)])
        plsc.subcore_barrier()

    def blk(bi, carry):
        pltpu.sync_copy(src2_hbm.at[pl.ds(rbase + bi * IB, IB)], sidx)
        pltpu.sync_copy(dst2_hbm.at[pl.ds(rbase + bi * IB, IB)], didx)
        pltpu.async_copy(h_hbm.at[sidx.at[0]], rows0, g0)

        def step2(i2, carry2):
            i0 = 2 * i2
            pltpu.make_async_copy(h_hbm.at[sidx.at[i0]], rows0, g0).wait()
            pltpu.async_copy(rows0, acc.at[didx.at[i0]], s0, add=True)

            @pl.when(i2 > 0)
            def _():
                pltpu.make_async_copy(rows1, acc.at[didx.at[i0 - 1]], s1).wait()

            pltpu.async_copy(h_hbm.at[sidx.at[i0 + 1]], rows1, g1)
            pltpu.make_async_copy(h_hbm.at[sidx.at[i0 + 1]], rows1, g1).wait()
            pltpu.async_copy(rows1, acc.at[didx.at[i0 + 1]], s1, add=True)
            pltpu.make_async_copy(rows0, acc.at[didx.at[i0]], s0).wait()

            @pl.when(i0 + 2 < IB)
            def _():
                pltpu.async_copy(h_hbm.at[sidx.at[i0 + 2]], rows0, g0)

            return carry2

        lax.fori_loop(0, IB // 2, step2, 0)
        pltpu.make_async_copy(rows1, acc.at[didx.at[IB - 1]], s1).wait()
        return carry

    with jax.named_scope("chunks"):
        lax.fori_loop(0, nblk, blk, 0)
        plsc.subcore_barrier()
    with jax.named_scope("writeback"):
        for j in range(RPT // K):
            pltpu.sync_copy(acc.at[pl.ds(s * RPT + j * K, K)], rows0)
            pltpu.sync_copy(
                rows0, out_acc.at[pl.ds(c * NP + s * RPT + j * K, K)])


_edge_kernel = pl.kernel(
    _edge_body,
    out_type=jax.ShapeDtypeStruct((NC * NP, D), jnp.float32),
    mesh=_MESH,
    scratch_types=[
        pltpu.VMEM_SHARED((NP, D), jnp.float32),
        pltpu.VMEM((IB, K), jnp.int32),
        pltpu.VMEM((IB, K), jnp.int32),
        pltpu.VMEM((K, D), jnp.float32),
        pltpu.VMEM((K, D), jnp.float32),
        pltpu.SemaphoreType.DMA,
        pltpu.SemaphoreType.DMA,
        pltpu.SemaphoreType.DMA,
        pltpu.SemaphoreType.DMA,
    ],
)


def _scale_body(f_ref, h0_ref, h1_ref, o_ref):
    cnt = h0_ref[0, 0, :] + h1_ref[0, 0, :]
    scale = lax.rsqrt(jnp.maximum(cnt, 1.0))
    o_ref[...] = f_ref[...] * scale[:, None]


def _out_body(a0_ref, a1_ref, h0_ref, h1_ref, w_ref, b_ref, o_ref):
    cnt = h0_ref[0, 0, :] + h1_ref[0, 0, :]
    inv = lax.rsqrt(jnp.maximum(cnt, 1.0))
    x = (a0_ref[...] + a1_ref[...]) * inv[:, None]
    y = jnp.dot(x, w_ref[...], preferred_element_type=jnp.float32)
    o_ref[...] = jnp.maximum(y + b_ref[0:1, :], 0.0)


def kernel(feature, edge_index, W, b):
    src = edge_index[0]
    dst = edge_index[1]
    pad = jnp.full((E_PAD - E,), NP - 1, dtype=jnp.int32)
    src2 = jnp.concatenate([src, pad]).reshape(E_PAD // K, K)
    dst2 = jnp.concatenate([dst, pad]).reshape(E_PAD // K, K)
    feature_p = jnp.pad(feature, ((0, NP - N), (0, 0)))
    zer_h = jnp.zeros((HB, 128), dtype=jnp.float32)
    zer_r = jnp.zeros((K, D), dtype=jnp.float32)
    iden = jnp.arange(HB, dtype=jnp.int32)
    b2 = jnp.broadcast_to(b, (8, D))

    hist_s, hist_d = _deg_kernel(src2, dst2, zer_h, iden)
    hist_s3 = hist_s.reshape(NC * GB, 1, 1024)
    hist_d3 = hist_d.reshape(NC * GB, 1, 1024)

    h = pl.pallas_call(
        _scale_body,
        grid=(GB,),
        in_specs=[pl.BlockSpec((1024, D), lambda i: (i, 0)),
                  pl.BlockSpec((1, 1, 1024), lambda i: (i, 0, 0)),
                  pl.BlockSpec((1, 1, 1024), lambda i: (i + GB, 0, 0))],
        out_specs=pl.BlockSpec((1024, D), lambda i: (i, 0)),
        out_shape=jax.ShapeDtypeStruct((NP, D), jnp.float32),
    )(feature_p, hist_s3, hist_s3)

    acc = _edge_kernel(src2, dst2, h, zer_r)

    out = pl.pallas_call(
        _out_body,
        grid=(GB,),
        in_specs=[pl.BlockSpec((1024, D), lambda i: (i, 0)),
                  pl.BlockSpec((1024, D), lambda i: (i + GB, 0)),
                  pl.BlockSpec((1, 1, 1024), lambda i: (i, 0, 0)),
                  pl.BlockSpec((1, 1, 1024), lambda i: (i + GB, 0, 0)),
                  pl.BlockSpec((128, D), lambda i: (0, 0)),
                  pl.BlockSpec((8, D), lambda i: (0, 0))],
        out_specs=pl.BlockSpec((1024, D), lambda i: (i, 0)),
        out_shape=jax.ShapeDtypeStruct((NP, D), jnp.float32),
    )(acc, acc, hist_d3, hist_d3, W, b2)

    return out[:N]


# speedup vs baseline: 4.1929x; 1.0134x over previous
import jax
import jax.numpy as jnp
from jax import lax
from jax.experimental import pallas as pl
from jax.experimental.pallas import tpu as pltpu
from jax.experimental.pallas import tpu_sc as plsc

N = 10000
E = 320000
D = 128
NC, NS = 2, 16
K = 128
CPW0 = 144
CPW1 = 16
IB = 16
DCW0 = 112
DCW1 = 48
E_PAD = NS * (CPW0 + CPW1) * K
NP = 10240
RPT = NP // NS
HB = NP // K
GB = NP // 1024

_MESH = plsc.VectorSubcoreMesh(
    core_axis_name="c", subcore_axis_name="s", num_cores=NC, num_subcores=NS)


def _worker_layout(c, s):
    rbase = jnp.where(c == 0, s * CPW0, NS * CPW0 + s * CPW1)
    nblk = jnp.where(c == 0, CPW0 // IB, CPW1 // IB)
    return rbase, nblk


def _deg_body(src2_hbm, dst2_hbm, zer2_hbm, iden_hbm, out_s, out_d,
              sh_s, sh_d, hs_v, hd_v, sidx, didx, iden_v, tbuf):
    c = lax.axis_index("c")
    s = lax.axis_index("s")
    rbase = jnp.where(c == 0, s * DCW0, NS * DCW0 + s * DCW1)
    nblk = jnp.where(c == 0, DCW0 // IB, DCW1 // IB)
    pltpu.sync_copy(zer2_hbm, hs_v)
    pltpu.sync_copy(zer2_hbm, hd_v)
    pltpu.sync_copy(iden_hbm, iden_v)

    @pl.when(s < HB // 8)
    def _():
        pltpu.sync_copy(hs_v.at[pl.ds(s * 8, 8)], sh_s.at[pl.ds(s * 8, 8)])
        pltpu.sync_copy(hd_v.at[pl.ds(s * 8, 8)], sh_d.at[pl.ds(s * 8, 8)])

    ones = jnp.ones((16,), jnp.float32)

    def blk(bi, carry):
        pltpu.sync_copy(src2_hbm.at[pl.ds(rbase + bi * IB, IB)], sidx)
        pltpu.sync_copy(dst2_hbm.at[pl.ds(rbase + bi * IB, IB)], didx)

        def step(i, carry2):
            for j in range(K // 16):
                si = sidx[i, pl.ds(j * 16, 16)]
                plsc.addupdate_scatter(
                    hs_v,
                    [lax.shift_right_logical(si, 7), lax.bitwise_and(si, 127)],
                    ones)
                di = didx[i, pl.ds(j * 16, 16)]
                plsc.addupdate_scatter(
                    hd_v,
                    [lax.shift_right_logical(di, 7), lax.bitwise_and(di, 127)],
                    ones)
            return carry2

        lax.fori_loop(0, IB, step, 0)
        return carry

    lax.fori_loop(0, nblk, blk, 0)
    plsc.subcore_barrier()
    pltpu.sync_copy(hs_v, sh_s.at[iden_v], add=True)
    pltpu.sync_copy(hd_v, sh_d.at[iden_v], add=True)
    plsc.subcore_barrier()

    @pl.when(s < HB // 8)
    def _():
        pltpu.sync_copy(sh_s.at[pl.ds(s * 8, 8)], tbuf)
        pltpu.sync_copy(tbuf, out_s.at[pl.ds(c * HB + s * 8, 8)])
        pltpu.sync_copy(sh_d.at[pl.ds(s * 8, 8)], tbuf)
        pltpu.sync_copy(tbuf, out_d.at[pl.ds(c * HB + s * 8, 8)])


_deg_kernel = pl.kernel(
    _deg_body,
    out_type=[jax.ShapeDtypeStruct((NC * HB, 128), jnp.float32),
              jax.ShapeDtypeStruct((NC * HB, 128), jnp.float32)],
    mesh=_MESH,
    scratch_types=[
        pltpu.VMEM_SHARED((HB, 128), jnp.float32),
        pltpu.VMEM_SHARED((HB, 128), jnp.float32),
        pltpu.VMEM((HB, 128), jnp.float32),
        pltpu.VMEM((HB, 128), jnp.float32),
        pltpu.VMEM((IB, K), jnp.int32),
        pltpu.VMEM((IB, K), jnp.int32),
        pltpu.VMEM((HB,), jnp.int32),
        pltpu.VMEM((8, 128), jnp.float32),
    ],
    compiler_params=pltpu.CompilerParams(needs_layout_passes=False),
)


def _edge_body(src2_hbm, dst2_hbm, h_hbm, zer2_hbm, out_acc,
               acc, sidx, didx, rows0, rows1, g0, g1):
    c = lax.axis_index("c")
    s = lax.axis_index("s")
    rbase, nblk = _worker_layout(c, s)
    with jax.named_scope("zero_acc"):
        pltpu.sync_copy(zer2_hbm, rows0)
        for j in range(RPT // K):
            pltpu.sync_copy(rows0, acc.at[pl.ds(s * RPT + j * K, K)])
        plsc.subcore_barrier()

    def blk(bi, carry):
        pltpu.sync_copy(src2_hbm.at[pl.ds(rbase + bi * IB, IB)], sidx)
        pltpu.sync_copy(dst2_hbm.at[pl.ds(rbase + bi * IB, IB)], didx)
        pltpu.async_copy(h_hbm.at[sidx.at[0]], rows0, g0)

        def step2(i2, carry2):
            i0 = 2 * i2
            pltpu.async_copy(h_hbm.at[sidx.at[i0 + 1]], rows1, g1)
            pltpu.make_async_copy(h_hbm.at[sidx.at[i0]], rows0, g0).wait()
            pltpu.sync_copy(rows0, acc.at[didx.at[i0]], add=True)

            @pl.when(i0 + 2 < IB)
            def _():
                pltpu.async_copy(h_hbm.at[sidx.at[i0 + 2]], rows0, g0)

            pltpu.make_async_copy(h_hbm.at[sidx.at[i0 + 1]], rows1, g1).wait()
            pltpu.sync_copy(rows1, acc.at[didx.at[i0 + 1]], add=True)
            return carry2

        lax.fori_loop(0, IB // 2, step2, 0)
        return carry

    with jax.named_scope("chunks"):
        lax.fori_loop(0, nblk, blk, 0)
        plsc.subcore_barrier()
    with jax.named_scope("writeback"):
        for j in range(RPT // K):
            pltpu.sync_copy(acc.at[pl.ds(s * RPT + j * K, K)], rows0)
            pltpu.sync_copy(
                rows0, out_acc.at[pl.ds(c * NP + s * RPT + j * K, K)])


_edge_kernel = pl.kernel(
    _edge_body,
    out_type=jax.ShapeDtypeStruct((NC * NP, D), jnp.float32),
    mesh=_MESH,
    scratch_types=[
        pltpu.VMEM_SHARED((NP, D), jnp.float32),
        pltpu.VMEM((IB, K), jnp.int32),
        pltpu.VMEM((IB, K), jnp.int32),
        pltpu.VMEM((K, D), jnp.float32),
        pltpu.VMEM((K, D), jnp.float32),
        pltpu.SemaphoreType.DMA,
        pltpu.SemaphoreType.DMA,
    ],
)


def _scale_body(f_ref, h0_ref, h1_ref, o_ref):
    cnt = h0_ref[0, 0, :] + h1_ref[0, 0, :]
    scale = lax.rsqrt(jnp.maximum(cnt, 1.0))
    o_ref[...] = f_ref[...] * scale[:, None]


def _out_body(a0_ref, a1_ref, h0_ref, h1_ref, w_ref, b_ref, o_ref):
    cnt = h0_ref[0, 0, :] + h1_ref[0, 0, :]
    inv = lax.rsqrt(jnp.maximum(cnt, 1.0))
    x = (a0_ref[...] + a1_ref[...]) * inv[:, None]
    y = jnp.dot(x, w_ref[...], preferred_element_type=jnp.float32)
    o_ref[...] = jnp.maximum(y + b_ref[0:1, :], 0.0)


def kernel(feature, edge_index, W, b):
    src = edge_index[0]
    dst = edge_index[1]
    pad = jnp.full((E_PAD - E,), NP - 1, dtype=jnp.int32)
    src2 = jnp.concatenate([src, pad]).reshape(E_PAD // K, K)
    dst2 = jnp.concatenate([dst, pad]).reshape(E_PAD // K, K)
    feature_p = jnp.pad(feature, ((0, NP - N), (0, 0)))
    zer_h = jnp.zeros((HB, 128), dtype=jnp.float32)
    zer_r = jnp.zeros((K, D), dtype=jnp.float32)
    iden = jnp.arange(HB, dtype=jnp.int32)
    b2 = jnp.broadcast_to(b, (8, D))

    hist_s, hist_d = _deg_kernel(src2, dst2, zer_h, iden)
    hist_s3 = hist_s.reshape(NC * GB, 1, 1024)
    hist_d3 = hist_d.reshape(NC * GB, 1, 1024)

    h = pl.pallas_call(
        _scale_body,
        grid=(GB,),
        in_specs=[pl.BlockSpec((1024, D), lambda i: (i, 0)),
                  pl.BlockSpec((1, 1, 1024), lambda i: (i, 0, 0)),
                  pl.BlockSpec((1, 1, 1024), lambda i: (i + GB, 0, 0))],
        out_specs=pl.BlockSpec((1024, D), lambda i: (i, 0)),
        out_shape=jax.ShapeDtypeStruct((NP, D), jnp.float32),
    )(feature_p, hist_s3, hist_s3)

    acc = _edge_kernel(src2, dst2, h, zer_r)

    out = pl.pallas_call(
        _out_body,
        grid=(GB,),
        in_specs=[pl.BlockSpec((1024, D), lambda i: (i, 0)),
                  pl.BlockSpec((1024, D), lambda i: (i + GB, 0)),
                  pl.BlockSpec((1, 1, 1024), lambda i: (i, 0, 0)),
                  pl.BlockSpec((1, 1, 1024), lambda i: (i + GB, 0, 0)),
                  pl.BlockSpec((128, D), lambda i: (0, 0)),
                  pl.BlockSpec((8, D), lambda i: (0, 0))],
        out_specs=pl.BlockSpec((1024, D), lambda i: (i, 0)),
        out_shape=jax.ShapeDtypeStruct((NP, D), jnp.float32),
    )(acc, acc, hist_d3, hist_d3, W, b2)

    return out[:N]
